# baseline (speedup 1.0000x reference)
import functools

import jax
import jax.numpy as jnp
from jax import lax
from jax.experimental import pallas as pl
from jax.experimental.pallas import tpu as pltpu
from jax.experimental.pallas import tpu_sc as plsc

_BATCH = 4
_SEQ = 8192
_EMB = 128
_NUM_CORES = 2
_NUM_SUBCORES = 16
_NUM_WORKERS = _NUM_CORES * _NUM_SUBCORES
_ROWS_PER_WORKER = _BATCH * _SEQ // _NUM_WORKERS
_WORKERS_PER_BATCH = _SEQ // _ROWS_PER_WORKER
_CHUNK = 128
_NUM_CHUNKS = _ROWS_PER_WORKER // _CHUNK
_LANES = 16
_NBUF = 3


@functools.partial(
    pl.kernel,
    out_type=jax.ShapeDtypeStruct((_BATCH, _SEQ, _EMB), jnp.float32),
    mesh=plsc.VectorSubcoreMesh(
        core_axis_name="c",
        subcore_axis_name="s",
        num_cores=_NUM_CORES,
        num_subcores=_NUM_SUBCORES,
    ),
    scratch_types=[
        pltpu.VMEM((_ROWS_PER_WORKER,), jnp.int32),
        pltpu.VMEM((_ROWS_PER_WORKER,), jnp.int32),
    ]
    + [pltpu.VMEM((_CHUNK, _EMB), jnp.float32) for _ in range(2 * _NBUF)]
    + [pltpu.SemaphoreType.DMA for _ in range(3 * _NBUF)],
)
def _embed_sc(tok_ids_hbm, pos_ids_hbm, tok_tab_hbm, pos_tab_hbm, out_hbm,
              idx_t, idx_p,
              rt0, rt1, rt2, rp0, rp1, rp2,
              sgt0, sgt1, sgt2, sgp0, sgp1, sgp2, swb0, swb1, swb2):
    rt = (rt0, rt1, rt2)
    rp = (rp0, rp1, rp2)
    sgt = (sgt0, sgt1, sgt2)
    sgp = (sgp0, sgp1, sgp2)
    swb = (swb0, swb1, swb2)

    wid = lax.axis_index("s") * _NUM_CORES + lax.axis_index("c")
    bi = wid // _WORKERS_PER_BATCH
    soff = (wid % _WORKERS_PER_BATCH) * _ROWS_PER_WORKER

    pltpu.sync_copy(tok_ids_hbm.at[bi, pl.ds(soff, _ROWS_PER_WORKER)], idx_t)
    pltpu.sync_copy(pos_ids_hbm.at[bi, pl.ds(soff, _ROWS_PER_WORKER)], idx_p)

    gt = [None] * _NBUF
    gp = [None] * _NBUF
    wb = [None] * _NBUF

    def fire(ci):
        b = ci % _NBUF
        sl = pl.ds(ci * _CHUNK, _CHUNK)
        gt[b] = pltpu.async_copy(tok_tab_hbm.at[idx_t.at[sl]], rt[b], sgt[b])
        gp[b] = pltpu.async_copy(pos_tab_hbm.at[idx_p.at[sl]], rp[b], sgp[b])

    for ci in range(min(_NBUF - 1, _NUM_CHUNKS)):
        fire(ci)

    for ci in range(_NUM_CHUNKS):
        b = ci % _NBUF
        gt[b].wait()
        gp[b].wait()

        rtb, rpb = rt[b], rp[b]

        def row_body(r, c):
            for j in range(_EMB // _LANES):
                sl = pl.ds(j * _LANES, _LANES)
                plsc.addupdate(rtb.at[r, sl], rpb[r, sl])
            return c

        lax.fori_loop(0, _CHUNK, row_body, 0, unroll=2)

        dst = out_hbm.at[bi, pl.ds(soff + ci * _CHUNK, _CHUNK)]
        wb[b] = pltpu.async_copy(rt[b], dst, swb[b])

        nxt = ci + _NBUF - 1
        if nxt < _NUM_CHUNKS:
            if nxt >= _NBUF:
                wb[nxt % _NBUF].wait()
            fire(nxt)

    for ci in range(max(0, _NUM_CHUNKS - _NBUF), _NUM_CHUNKS):
        wb[ci % _NBUF].wait()


def kernel(token_ids, position_ids, tok_table, pos_table):
    return _embed_sc(token_ids.astype(jnp.int32), position_ids.astype(jnp.int32),
                     tok_table, pos_table)


# speedup vs baseline: 1.0264x; 1.0264x over previous
import functools

import jax
import jax.numpy as jnp
from jax import lax
from jax.experimental import pallas as pl
from jax.experimental.pallas import tpu as pltpu
from jax.experimental.pallas import tpu_sc as plsc

_BATCH = 4
_SEQ = 8192
_EMB = 128
_NUM_CORES = 2
_NUM_SUBCORES = 16
_NUM_WORKERS = _NUM_CORES * _NUM_SUBCORES
_ROWS_PER_WORKER = _BATCH * _SEQ // _NUM_WORKERS
_WORKERS_PER_BATCH = _SEQ // _ROWS_PER_WORKER
_CHUNK = 128
_NUM_CHUNKS = _ROWS_PER_WORKER // _CHUNK
_NBUF = 4


@functools.partial(
    pl.kernel,
    out_type=jax.ShapeDtypeStruct((_BATCH, _SEQ, _EMB), jnp.float32),
    mesh=plsc.VectorSubcoreMesh(
        core_axis_name="c",
        subcore_axis_name="s",
        num_cores=_NUM_CORES,
        num_subcores=_NUM_SUBCORES,
    ),
    scratch_types=[
        pltpu.VMEM((_ROWS_PER_WORKER,), jnp.int32),
        pltpu.VMEM((_ROWS_PER_WORKER,), jnp.int32),
    ]
    + [pltpu.VMEM((_CHUNK, _EMB), jnp.float32) for _ in range(_NBUF)]
    + [pltpu.SemaphoreType.DMA for _ in range(3 * _NBUF)],
)
def _embed_sc(tok_ids_hbm, pos_ids_hbm, tok_tab_hbm, pos_tab_hbm, out_hbm,
              idx_t, idx_p,
              r0, r1, r2, r3,
              sgt0, sgt1, sgt2, sgt3, sgp0, sgp1, sgp2, sgp3,
              swb0, swb1, swb2, swb3):
    rows = (r0, r1, r2, r3)
    sgt = (sgt0, sgt1, sgt2, sgt3)
    sgp = (sgp0, sgp1, sgp2, sgp3)
    swb = (swb0, swb1, swb2, swb3)

    wid = lax.axis_index("s") * _NUM_CORES + lax.axis_index("c")
    bi = wid // _WORKERS_PER_BATCH
    soff = (wid % _WORKERS_PER_BATCH) * _ROWS_PER_WORKER

    pltpu.sync_copy(tok_ids_hbm.at[bi, pl.ds(soff, _ROWS_PER_WORKER)], idx_t)
    pltpu.sync_copy(pos_ids_hbm.at[bi, pl.ds(soff, _ROWS_PER_WORKER)], idx_p)

    gt = [None] * _NBUF
    gp = [None] * _NBUF
    wb = [None] * _NBUF

    def fire_tok(ci):
        b = ci % _NBUF
        sl = pl.ds(ci * _CHUNK, _CHUNK)
        gt[b] = pltpu.async_copy(tok_tab_hbm.at[idx_t.at[sl]], rows[b], sgt[b])

    def fire_pos_add(ci):
        b = ci % _NBUF
        sl = pl.ds(ci * _CHUNK, _CHUNK)
        gp[b] = pltpu.async_copy(pos_tab_hbm.at[idx_p.at[sl]], rows[b], sgp[b],
                                 add=True)

    fire_tok(0)
    gt[0].wait()
    fire_pos_add(0)
    if _NUM_CHUNKS > 1:
        fire_tok(1)

    for ci in range(_NUM_CHUNKS):
        b = ci % _NBUF
        gp[b].wait()

        dst = out_hbm.at[bi, pl.ds(soff + ci * _CHUNK, _CHUNK)]
        wb[b] = pltpu.async_copy(rows[b], dst, swb[b])

        if ci + 1 < _NUM_CHUNKS:
            gt[(ci + 1) % _NBUF].wait()
            fire_pos_add(ci + 1)
        if ci + 2 < _NUM_CHUNKS:
            b2 = (ci + 2) % _NBUF
            if ci + 2 >= _NBUF:
                wb[b2].wait()
            fire_tok(ci + 2)

    for ci in range(max(0, _NUM_CHUNKS - _NBUF), _NUM_CHUNKS):
        wb[ci % _NBUF].wait()


def kernel(token_ids, position_ids, tok_table, pos_table):
    return _embed_sc(token_ids.astype(jnp.int32), position_ids.astype(jnp.int32),
                     tok_table, pos_table)


# speedup vs baseline: 1.0305x; 1.0040x over previous
import functools

import jax
import jax.numpy as jnp
from jax import lax
from jax.experimental import pallas as pl
from jax.experimental.pallas import tpu as pltpu
from jax.experimental.pallas import tpu_sc as plsc

_BATCH = 4
_SEQ = 8192
_EMB = 128
_NUM_CORES = 2
_NUM_SUBCORES = 16
_NUM_WORKERS = _NUM_CORES * _NUM_SUBCORES
_ROWS_PER_WORKER = _BATCH * _SEQ // _NUM_WORKERS
_WORKERS_PER_BATCH = _SEQ // _ROWS_PER_WORKER
_CHUNK = 128
_NUM_CHUNKS = _ROWS_PER_WORKER // _CHUNK
_NBUF = 6
_TOK_AHEAD = 4


@functools.partial(
    pl.kernel,
    out_type=jax.ShapeDtypeStruct((_BATCH, _SEQ, _EMB), jnp.float32),
    mesh=plsc.VectorSubcoreMesh(
        core_axis_name="c",
        subcore_axis_name="s",
        num_cores=_NUM_CORES,
        num_subcores=_NUM_SUBCORES,
    ),
    scratch_types=[
        pltpu.VMEM((_ROWS_PER_WORKER,), jnp.int32),
        pltpu.VMEM((_ROWS_PER_WORKER,), jnp.int32),
    ]
    + [pltpu.VMEM((_CHUNK, _EMB), jnp.float32) for _ in range(_NBUF)]
    + [pltpu.SemaphoreType.DMA for _ in range(3 * _NBUF + 2)],
)
def _embed_sc(tok_ids_hbm, pos_ids_hbm, tok_tab_hbm, pos_tab_hbm, out_hbm,
              idx_t, idx_p,
              r0, r1, r2, r3, r4, r5,
              sgt0, sgt1, sgt2, sgt3, sgt4, sgt5,
              sgp0, sgp1, sgp2, sgp3, sgp4, sgp5,
              swb0, swb1, swb2, swb3, swb4, swb5,
              sid_t, sid_p):
    rows = (r0, r1, r2, r3, r4, r5)
    sgt = (sgt0, sgt1, sgt2, sgt3, sgt4, sgt5)
    sgp = (sgp0, sgp1, sgp2, sgp3, sgp4, sgp5)
    swb = (swb0, swb1, swb2, swb3, swb4, swb5)

    wid = lax.axis_index("s") * _NUM_CORES + lax.axis_index("c")
    bi = wid // _WORKERS_PER_BATCH
    soff = (wid % _WORKERS_PER_BATCH) * _ROWS_PER_WORKER

    cid_t = pltpu.async_copy(
        tok_ids_hbm.at[bi, pl.ds(soff, _ROWS_PER_WORKER)], idx_t, sid_t)
    cid_p = pltpu.async_copy(
        pos_ids_hbm.at[bi, pl.ds(soff, _ROWS_PER_WORKER)], idx_p, sid_p)
    cid_t.wait()
    cid_p.wait()

    gt = [None] * _NBUF
    gp = [None] * _NBUF
    wb = [None] * _NBUF

    def fire_tok(ci):
        b = ci % _NBUF
        sl = pl.ds(ci * _CHUNK, _CHUNK)
        gt[b] = pltpu.async_copy(tok_tab_hbm.at[idx_t.at[sl]], rows[b], sgt[b])

    def fire_pos_add(ci):
        b = ci % _NBUF
        sl = pl.ds(ci * _CHUNK, _CHUNK)
        gp[b] = pltpu.async_copy(pos_tab_hbm.at[idx_p.at[sl]], rows[b], sgp[b],
                                 add=True)

    for ci in range(min(_TOK_AHEAD, _NUM_CHUNKS)):
        fire_tok(ci)
    gt[0].wait()
    fire_pos_add(0)

    for ci in range(_NUM_CHUNKS):
        b = ci % _NBUF
        gp[b].wait()

        dst = out_hbm.at[bi, pl.ds(soff + ci * _CHUNK, _CHUNK)]
        wb[b] = pltpu.async_copy(rows[b], dst, swb[b])

        if ci + 1 < _NUM_CHUNKS:
            gt[(ci + 1) % _NBUF].wait()
            fire_pos_add(ci + 1)

        nt = ci + _TOK_AHEAD
        if nt < _NUM_CHUNKS:
            bn = nt % _NBUF
            if nt >= _NBUF:
                wb[bn].wait()
            fire_tok(nt)

    for ci in range(max(0, _NUM_CHUNKS - _NBUF), _NUM_CHUNKS):
        wb[ci % _NBUF].wait()


def kernel(token_ids, position_ids, tok_table, pos_table):
    return _embed_sc(token_ids.astype(jnp.int32), position_ids.astype(jnp.int32),
                     tok_table, pos_table)
